# baseline (speedup 1.0000x reference)
import jax
import jax.numpy as jnp
from jax.experimental import pallas as pl

N_GROUPS = 1024
GROUP_RANK = 16
K_GROUPS = 64


def kernel(x, W_enc, W_dec, b_enc, b_dec):
    pre_acts = jnp.dot(x.astype(jnp.bfloat16), W_enc.astype(jnp.bfloat16),
                       preferred_element_type=jnp.float32) + b_enc
    groups = pre_acts.reshape(-1, N_GROUPS, GROUP_RANK)
    group_norms = jnp.sqrt(jnp.sum(jnp.square(groups.astype(jnp.float32)), axis=-1))
    _, topk_idx = jax.lax.top_k(group_norms, K_GROUPS)
    bsz = group_norms.shape[0]
    mask = jnp.zeros((bsz, N_GROUPS), dtype=jnp.float32)
    mask = mask.at[jnp.arange(bsz)[:, None], topk_idx].set(1.0)
    active_groups = groups * mask[..., None]
    feature_acts = active_groups.reshape(-1, N_GROUPS * GROUP_RANK)
    sae_out = jnp.dot(feature_acts.astype(jnp.bfloat16), W_dec.astype(jnp.bfloat16),
                      preferred_element_type=jnp.float32) + b_dec
    return sae_out


# speedup vs baseline: 2.2478x; 2.2478x over previous
import functools

import jax
import jax.numpy as jnp
from jax.experimental import pallas as pl
from jax.experimental.pallas import tpu as pltpu

_GROUP_RANK = 16
_K_GROUPS = 64
_T = 256
_FB = 1024


def _fused(x_ref, we_ref, wd_ref, be_ref, bd_ref, out_ref,
           pre_ref, nrm_ref, msk_ref, *, nfb, gpb):
    p = pl.program_id(1)
    t_tile = pre_ref.shape[1]

    @pl.when(p < nfb)
    def _encode():
        blk = jnp.dot(x_ref[...], we_ref[...],
                      preferred_element_type=jnp.float32)
        blk = blk + be_ref[p]
        pre_ref[p] = blk
        sq = blk * blk
        hi = sq.astype(jnp.bfloat16)
        lo = (sq - hi.astype(jnp.float32)).astype(jnp.bfloat16)
        g_ind = (jax.lax.broadcasted_iota(jnp.int32, (_FB, gpb), 0)
                 // _GROUP_RANK
                 == jax.lax.broadcasted_iota(jnp.int32, (_FB, gpb), 1)
                 ).astype(jnp.bfloat16)
        nrm_ref[p] = (jnp.dot(hi, g_ind, preferred_element_type=jnp.float32)
                      + jnp.dot(lo, g_ind, preferred_element_type=jnp.float32))

    @pl.when(p == nfb)
    def _select():
        bits = jax.lax.bitcast_convert_type(nrm_ref[...], jnp.int32)

        def body(_, carry):
            lo_b, hi_b = carry
            mid = lo_b + ((hi_b - lo_b) >> 1)
            cnt = jnp.sum((bits >= mid).astype(jnp.int32), axis=(0, 2),
                          keepdims=True)
            ok = cnt >= _K_GROUPS
            return jnp.where(ok, mid, lo_b), jnp.where(ok, hi_b, mid)

        lo0 = jnp.zeros((1, t_tile, 1), jnp.int32)
        hi0 = jnp.full((1, t_tile, 1), jnp.int32(0x7F800000))
        thr, _ = jax.lax.fori_loop(0, 31, body, (lo0, hi0))
        msk_ref[...] = (bits >= thr).astype(jnp.bfloat16)

    @pl.when(p >= nfb)
    def _decode():
        b = p - nfb
        gt_ind = (jax.lax.broadcasted_iota(jnp.int32, (gpb, _FB), 0)
                  == jax.lax.broadcasted_iota(jnp.int32, (gpb, _FB), 1)
                  // _GROUP_RANK).astype(jnp.bfloat16)
        mfeat = jnp.dot(msk_ref[b], gt_ind,
                        preferred_element_type=jnp.float32
                        ).astype(jnp.bfloat16)
        masked = pre_ref[b].astype(jnp.bfloat16) * mfeat
        acc = jnp.dot(masked, wd_ref[...], preferred_element_type=jnp.float32)

        @pl.when(b == 0)
        def _init():
            out_ref[...] = acc + bd_ref[...]

        @pl.when(b > 0)
        def _acc():
            out_ref[...] += acc


def kernel(x, W_enc, W_dec, b_enc, b_dec):
    tokens, d_model = x.shape
    d_sae = W_enc.shape[1]
    nfb = d_sae // _FB
    gpb = _FB // _GROUP_RANK

    x16 = x.astype(jnp.bfloat16)
    we16 = W_enc.astype(jnp.bfloat16)
    wd16 = W_dec.astype(jnp.bfloat16)
    be3 = b_enc.reshape(nfb, 1, _FB)
    bd2 = b_dec.reshape(1, d_model)

    grid = (tokens // _T, 2 * nfb)
    body = functools.partial(_fused, nfb=nfb, gpb=gpb)
    return pl.pallas_call(
        body,
        grid=grid,
        in_specs=[
            pl.BlockSpec((_T, d_model), lambda t, p: (t, 0)),
            pl.BlockSpec((d_model, _FB),
                         lambda t, p, n=nfb: (0, jnp.minimum(p, n - 1))),
            pl.BlockSpec((_FB, d_model),
                         lambda t, p, n=nfb: (jnp.maximum(p - n, 0), 0)),
            pl.BlockSpec((nfb, 1, _FB), lambda t, p: (0, 0, 0)),
            pl.BlockSpec((1, d_model), lambda t, p: (0, 0)),
        ],
        out_specs=pl.BlockSpec((_T, d_model), lambda t, p: (t, 0)),
        out_shape=jax.ShapeDtypeStruct((tokens, d_model), jnp.float32),
        scratch_shapes=[
            pltpu.VMEM((nfb, _T, _FB), jnp.float32),
            pltpu.VMEM((nfb, _T, gpb), jnp.float32),
            pltpu.VMEM((nfb, _T, gpb), jnp.bfloat16),
        ],
        compiler_params=pltpu.CompilerParams(
            dimension_semantics=("arbitrary", "arbitrary"),
        ),
    )(x16, we16, wd16, be3, bd2)
